# baseline (speedup 1.0000x reference)
import functools

import jax
import jax.numpy as jnp
from jax import lax
from jax.experimental import pallas as pl
from jax.experimental.pallas import tpu as pltpu
from jax.experimental.pallas import tpu_sc as plsc

N = 10000
NPAD = 10240
E = 320000
R = 16
D = 128
EPS = 1e-10

CHUNK = 80
NW = 32
EW = E // NW
ES = E // 16
NPS = NPAD // 16
HP = 4000
WP = 2000
SP = 2000

_mesh = plsc.VectorSubcoreMesh(core_axis_name="c", subcore_axis_name="s")


@functools.partial(
    pl.kernel,
    out_type=(
        jax.ShapeDtypeStruct((E,), jnp.float32),
        jax.ShapeDtypeStruct((E,), jnp.int32),
    ),
    mesh=_mesh,
    scratch_types=[
        pltpu.VMEM_SHARED((NPAD * 16,), jnp.float32),
        pltpu.VMEM((2048,), jnp.float32),
        pltpu.VMEM((HP,), jnp.int32),
        pltpu.VMEM((HP,), jnp.int32),
        pltpu.VMEM((CHUNK,), jnp.float32),
        pltpu.VMEM((CHUNK,), jnp.int32),
        pltpu.VMEM((CHUNK,), jnp.int32),
        pltpu.VMEM((CHUNK,), jnp.float32),
        pltpu.VMEM((CHUNK,), jnp.float32),
        pltpu.VMEM((WP,), jnp.int32),
        pltpu.VMEM((WP,), jnp.int32),
        pltpu.VMEM((WP,), jnp.int32),
        pltpu.VMEM((WP,), jnp.float32),
        pltpu.VMEM((WP,), jnp.int32),
        pltpu.SemaphoreType.DMA,
        pltpu.SemaphoreType.DMA,
    ],
)
def _prep(dst_hbm, rel_hbm, src_hbm, w_hbm, g_hbm,
          den, zb, dstb, relb, onesb, sega, segb, dena, denb,
          dst2, rel2, src2, wb, gb, sema, semb):
    c = lax.axis_index("c")
    s = lax.axis_index("s")
    wid = c * 16 + s
    zv = jnp.zeros((16,), jnp.float32)
    ones = jnp.ones((16,), jnp.float32)

    def zb_body(r, _):
        zb[pl.ds(r * 16, 16)] = zv
        return 0
    lax.fori_loop(0, 128, zb_body, 0)
    for k in range(CHUNK // 16):
        onesb[pl.ds(k * 16, 16)] = ones

    for k in range((NPAD * 16) // 2048 // 16):
        r0 = pl.multiple_of((s * 5 + k) * 2048, 8)
        pltpu.sync_copy(zb, den.at[pl.ds(r0, 2048)])
    plsc.subcore_barrier()

    def build(seg, srcd, srcr, i):
        for k in range(CHUNK // 16):
            ds16 = pl.ds(i * CHUNK + k * 16, 16)
            seg[pl.ds(k * 16, 16)] = srcd[ds16] * 16 + srcr[ds16]

    def addi(seg, sem):
        pltpu.async_copy(onesb, den.at[seg], sem, add=True)

    def addw(seg, sem):
        pltpu.make_async_copy(onesb, den.at[seg], sem).wait()

    HCH = HP // CHUNK

    def hist_phase(ph, _):
        base = pl.multiple_of(s * ES + ph * HP, 8)
        pltpu.sync_copy(dst_hbm.at[pl.ds(base, HP)], dstb)
        pltpu.sync_copy(rel_hbm.at[pl.ds(base, HP)], relb)
        build(sega, dstb, relb, 0)
        addi(sega, sema)

        def hist_pair(k, _):
            i0 = k * 2
            build(segb, dstb, relb, i0 + 1)
            addi(segb, semb)
            addw(sega, sema)
            build(sega, dstb, relb, i0 + 2)
            addi(sega, sema)
            addw(segb, semb)
            return 0
        lax.fori_loop(0, HCH // 2 - 1, hist_pair, 0)
        build(segb, dstb, relb, HCH - 1)
        addi(segb, semb)
        addw(sega, sema)
        addw(segb, semb)
        return 0
    lax.fori_loop(0, ES // HP, hist_phase, 0)
    plsc.subcore_barrier()

    WCH = WP // CHUNK

    def geti(seg, dvals, sem):
        pltpu.async_copy(den.at[seg], dvals, sem)

    def getw(seg, dvals, sem):
        pltpu.make_async_copy(den.at[seg], dvals, sem).wait()

    def w_phase(ph, _):
        base = pl.multiple_of(wid * EW + ph * WP, 8)
        pltpu.sync_copy(dst_hbm.at[pl.ds(base, WP)], dst2)
        pltpu.sync_copy(rel_hbm.at[pl.ds(base, WP)], rel2)
        pltpu.sync_copy(src_hbm.at[pl.ds(base, WP)], src2)

        def emit(dvals, i):
            for k in range(CHUNK // 16):
                ds16 = pl.ds(i * CHUNK + k * 16, 16)
                wb[ds16] = 1.0 / (dvals[pl.ds(k * 16, 16)] + EPS)
                gb[ds16] = rel2[ds16] * N + src2[ds16]

        build(sega, dst2, rel2, 0)
        geti(sega, dena, sema)

        def w_pair(k, _):
            i0 = k * 2
            build(segb, dst2, rel2, i0 + 1)
            geti(segb, denb, semb)
            getw(sega, dena, sema)
            emit(dena, i0)
            build(sega, dst2, rel2, i0 + 2)
            geti(sega, dena, sema)
            getw(segb, denb, semb)
            emit(denb, i0 + 1)
            return 0
        lax.fori_loop(0, WCH // 2, w_pair, 0)
        getw(sega, dena, sema)
        emit(dena, WCH - 1)

        pltpu.sync_copy(wb, w_hbm.at[pl.ds(base, WP)])
        pltpu.sync_copy(gb, g_hbm.at[pl.ds(base, WP)])
        return 0
    lax.fori_loop(0, EW // WP, w_phase, 0)


NCH = EW // CHUNK
HW = D // 2


@functools.partial(
    pl.kernel,
    out_type=jax.ShapeDtypeStruct((2, NPAD, D), jnp.float32),
    mesh=_mesh,
    scratch_types=[
        pltpu.VMEM_SHARED((NPAD, D), jnp.float32),
        pltpu.VMEM((CHUNK, HW), jnp.float32),
        pltpu.VMEM((CHUNK, HW), jnp.float32),
        pltpu.VMEM((CHUNK, D), jnp.float32),
        pltpu.VMEM((CHUNK, D), jnp.float32),
        pltpu.VMEM((CHUNK,), jnp.int32),
        pltpu.VMEM((CHUNK,), jnp.int32),
        pltpu.VMEM((CHUNK,), jnp.int32),
        pltpu.VMEM((CHUNK,), jnp.int32),
        pltpu.VMEM((CHUNK,), jnp.int32),
        pltpu.VMEM((CHUNK,), jnp.int32),
        pltpu.VMEM((CHUNK + 16,), jnp.float32),
        pltpu.VMEM((CHUNK + 16,), jnp.float32),
        pltpu.SemaphoreType.DMA,
        pltpu.SemaphoreType.DMA,
        pltpu.SemaphoreType.DMA,
        pltpu.SemaphoreType.DMA,
        pltpu.SemaphoreType.DMA,
        pltpu.SemaphoreType.DMA,
    ],
    compiler_params=pltpu.CompilerParams(needs_layout_passes=False,
                                         use_tc_tiling_on_sc=False),
)
def _scatter(z_hbm, g_hbm, dst_hbm, w_hbm, p_hbm,
             acc, pka, pkb, rowsa, rowsb, ga, gb, dca, dcb, sa, sb,
             wa, wb, mma, mmb, mga, mgb, msa, msb):
    c = lax.axis_index("c")
    s = lax.axis_index("s")
    wid = c * 16 + s
    zv = jnp.zeros((16,), jnp.float32)
    ziv = jnp.zeros((16,), jnp.int32)
    ebase = wid * EW
    MASK = jnp.full((16,), -65536, jnp.int32)

    A = (pka, rowsa, ga, dca, sa, wa, mma, mga, msa)
    B = (pkb, rowsb, gb, dcb, sb, wb, mmb, mgb, msb)

    def meta_issue(S, i):
        pk, rows, g, dc, sx, w, mm, mg, ms = S
        off = pl.multiple_of(ebase + i * CHUNK, 8)
        pltpu.async_copy(g_hbm.at[pl.ds(off, CHUNK)], g, mm)
        pltpu.async_copy(dst_hbm.at[pl.ds(off, CHUNK)], dc, mm)
        pltpu.async_copy(w_hbm.at[pl.ds(off, CHUNK)], w.at[pl.ds(0, CHUNK)], mm)

    def meta_wait(S, i):
        pk, rows, g, dc, sx, w, mm, mg, ms = S
        off = pl.multiple_of(ebase + i * CHUNK, 8)
        pltpu.make_async_copy(g_hbm.at[pl.ds(off, CHUNK)], g, mm).wait()
        pltpu.make_async_copy(dst_hbm.at[pl.ds(off, CHUNK)], dc, mm).wait()
        pltpu.make_async_copy(w_hbm.at[pl.ds(off, CHUNK)],
                              w.at[pl.ds(0, CHUNK)], mm).wait()

    def gather_issue(S):
        pk, rows, g, dc, sx, w, mm, mg, ms = S
        pltpu.async_copy(z_hbm.at[g], pk, mg)

    def gather_wait(S):
        pk, rows, g, dc, sx, w, mm, mg, ms = S
        pltpu.make_async_copy(z_hbm.at[g], pk, mg).wait()

    def sct_issue(S):
        pk, rows, g, dc, sx, w, mm, mg, ms = S
        pltpu.async_copy(rows, acc.at[sx], ms, add=True)

    def sct_wait(S):
        pk, rows, g, dc, sx, w, mm, mg, ms = S
        pltpu.make_async_copy(rows, acc.at[sx], ms).wait()

    def fillidx(S):
        pk, rows, g, dc, sx, w, mm, mg, ms = S
        for k in range(CHUNK // 16):
            sx[pl.ds(k * 16, 16)] = dc[pl.ds(k * 16, 16)]

    def scale(S):
        pk, rows, g, dc, sx, w, mm, mg, ms = S

        def edge_body(e, _):
            wv = w[pl.ds(e, 16)]
            ws = wv[0]
            for j in range(HW // 16):
                u = plsc.bitcast(pk[e, pl.ds(j * 16, 16)], jnp.int32)
                lo = plsc.bitcast(u << 16, jnp.float32)
                hi = plsc.bitcast(u & MASK, jnp.float32)
                rows[e, pl.ds(j * 16, 16)] = lo * ws
                rows[e, pl.ds(HW + j * 16, 16)] = hi * ws
            return 0
        lax.fori_loop(0, CHUNK, edge_body, 0, unroll=8)

    def zero_rows(rows):
        def zb_body(r, _):
            for j in range(D // 16):
                rows[r, pl.ds(j * 16, 16)] = zv
            return 0
        lax.fori_loop(0, CHUNK, zb_body, 0)

    zero_rows(rowsa)
    zero_rows(rowsb)
    for k in range(CHUNK // 16):
        sa[pl.ds(k * 16, 16)] = ziv
        sb[pl.ds(k * 16, 16)] = ziv
    for k in range(NPS // CHUNK):
        r0 = pl.multiple_of(s * NPS + k * CHUNK, 8)
        pltpu.sync_copy(rowsa, acc.at[pl.ds(r0, CHUNK)])
    plsc.subcore_barrier()

    sct_issue(A)
    sct_issue(B)
    meta_issue(A, 0)
    meta_issue(B, 1)
    meta_wait(A, 0)
    gather_issue(A)

    def step(P, Q, i, prefetch):
        meta_wait(Q, i + 1)
        gather_issue(Q)
        gather_wait(P)
        sct_wait(P)
        scale(P)
        fillidx(P)
        sct_issue(P)
        if prefetch:
            meta_issue(P, i + 2)

    def pair_body(k, _):
        i0 = k * 2
        step(A, B, i0, True)
        step(B, A, i0 + 1, True)
        return 0
    lax.fori_loop(0, (NCH - 3) // 2, pair_body, 0)

    j0 = NCH - 3
    step(A, B, j0, True)
    step(B, A, j0 + 1, False)
    gather_wait(A)
    sct_wait(A)
    scale(A)
    fillidx(A)
    sct_issue(A)
    sct_wait(B)
    sct_wait(A)
    plsc.subcore_barrier()

    for k in range(NPS // CHUNK):
        r0 = pl.multiple_of(s * NPS + k * CHUNK, 8)
        pltpu.sync_copy(acc.at[pl.ds(r0, CHUNK)], p_hbm.at[c, pl.ds(r0, CHUNK)])


def _zmm_body(h_ref, w_ref, z_ref):
    o = jnp.dot(h_ref[...], w_ref[...], preferred_element_type=jnp.float32)
    lo = jax.lax.bitcast_convert_type(
        o[:, :D // 2].astype(jnp.bfloat16), jnp.uint16).astype(jnp.uint32)
    hi = jax.lax.bitcast_convert_type(
        o[:, D // 2:].astype(jnp.bfloat16), jnp.uint16).astype(jnp.uint32)
    z_ref[0] = jax.lax.bitcast_convert_type((hi << 16) | lo, jnp.float32)


_zmm = pl.pallas_call(
    _zmm_body,
    grid=(R,),
    in_specs=[
        pl.BlockSpec((N, D), lambda r: (0, 0)),
        pl.BlockSpec((D, D), lambda r: (r, 0)),
    ],
    out_specs=pl.BlockSpec((1, N, D // 2), lambda r: (r, 0, 0)),
    out_shape=jax.ShapeDtypeStruct((R, N, D // 2), jnp.float32),
)


def _combine_body(p_ref, h_ref, ws_ref, b_ref, bs_ref, out_ref, gf_ref):
    rel_part = p_ref[0, :N, :] + p_ref[1, :N, :]
    o = rel_part + jnp.dot(h_ref[...], ws_ref[...],
                           preferred_element_type=jnp.float32)
    o = jnp.maximum(o + b_ref[...] + bs_ref[...], 0.0)
    out_ref[...] = o
    gf_ref[...] = jnp.sum(o, axis=0, keepdims=True)


_combine = pl.pallas_call(
    _combine_body,
    out_shape=(
        jax.ShapeDtypeStruct((N, D), jnp.float32),
        jax.ShapeDtypeStruct((1, D), jnp.float32),
    ),
)


def _layer(h, W, b, Ws, bs, g1, dst1, w1):
    z = _zmm(h, W).reshape(R * N, D // 2)
    p = _scatter(z, g1, dst1, w1)
    return _combine(p, h, Ws, b.reshape(1, D), bs.reshape(1, D))


def kernel(x, edge_index, edge_type, W1, b1, W1s, b1s, W2, b2, W2s, b2s):
    src = edge_index[0]
    dst = edge_index[1]
    rel = edge_type
    w1d, g1d = _prep(dst, rel, src)
    h1, _ = _layer(x, W1, b1, W1s, b1s, g1d, dst, w1d)
    h2, gf = _layer(h1, W2, b2, W2s, b2s, g1d, dst, w1d)
    return (gf, h2)


# speedup vs baseline: 2.1110x; 2.1110x over previous
import functools

import jax
import jax.numpy as jnp
from jax import lax
from jax.experimental import pallas as pl
from jax.experimental.pallas import tpu as pltpu
from jax.experimental.pallas import tpu_sc as plsc

N = 10000
NPAD = 10240
E = 320000
R = 16
D = 128
EPS = 1e-10

CHUNK = 80
NW = 32
EW = E // NW
ES = E // 16
NPS = NPAD // 16
HP = 4000
WP = 2000
SP = 2000

_mesh = plsc.VectorSubcoreMesh(core_axis_name="c", subcore_axis_name="s")


@functools.partial(
    pl.kernel,
    out_type=(
        jax.ShapeDtypeStruct((E,), jnp.float32),
        jax.ShapeDtypeStruct((E,), jnp.int32),
    ),
    mesh=_mesh,
    scratch_types=[
        pltpu.VMEM_SHARED((NPAD * 16,), jnp.float32),
        pltpu.VMEM((2048,), jnp.float32),
        pltpu.VMEM((HP,), jnp.int32),
        pltpu.VMEM((HP,), jnp.int32),
        pltpu.VMEM((CHUNK,), jnp.float32),
        pltpu.VMEM((CHUNK,), jnp.int32),
        pltpu.VMEM((CHUNK,), jnp.int32),
        pltpu.VMEM((CHUNK,), jnp.float32),
        pltpu.VMEM((CHUNK,), jnp.float32),
        pltpu.VMEM((WP,), jnp.int32),
        pltpu.VMEM((WP,), jnp.int32),
        pltpu.VMEM((WP,), jnp.int32),
        pltpu.VMEM((WP,), jnp.float32),
        pltpu.VMEM((WP,), jnp.int32),
        pltpu.SemaphoreType.DMA,
        pltpu.SemaphoreType.DMA,
    ],
)
def _prep(dst_hbm, rel_hbm, src_hbm, w_hbm, g_hbm,
          den, zb, dstb, relb, onesb, sega, segb, dena, denb,
          dst2, rel2, src2, wb, gb, sema, semb):
    c = lax.axis_index("c")
    s = lax.axis_index("s")
    wid = c * 16 + s
    zv = jnp.zeros((16,), jnp.float32)
    ones = jnp.ones((16,), jnp.float32)

    def zb_body(r, _):
        zb[pl.ds(r * 16, 16)] = zv
        return 0
    lax.fori_loop(0, 128, zb_body, 0)
    for k in range(CHUNK // 16):
        onesb[pl.ds(k * 16, 16)] = ones

    for k in range((NPAD * 16) // 2048 // 16):
        r0 = pl.multiple_of((s * 5 + k) * 2048, 8)
        pltpu.sync_copy(zb, den.at[pl.ds(r0, 2048)])
    plsc.subcore_barrier()

    def build(seg, srcd, srcr, i):
        for k in range(CHUNK // 16):
            ds16 = pl.ds(i * CHUNK + k * 16, 16)
            seg[pl.ds(k * 16, 16)] = srcd[ds16] * 16 + srcr[ds16]

    def addi(seg, sem):
        pltpu.async_copy(onesb, den.at[seg], sem, add=True)

    def addw(seg, sem):
        pltpu.make_async_copy(onesb, den.at[seg], sem).wait()

    HCH = HP // CHUNK

    def hist_phase(ph, _):
        base = pl.multiple_of(s * ES + ph * HP, 8)
        pltpu.sync_copy(dst_hbm.at[pl.ds(base, HP)], dstb)
        pltpu.sync_copy(rel_hbm.at[pl.ds(base, HP)], relb)
        build(sega, dstb, relb, 0)
        addi(sega, sema)

        def hist_pair(k, _):
            i0 = k * 2
            build(segb, dstb, relb, i0 + 1)
            addi(segb, semb)
            addw(sega, sema)
            build(sega, dstb, relb, i0 + 2)
            addi(sega, sema)
            addw(segb, semb)
            return 0
        lax.fori_loop(0, HCH // 2 - 1, hist_pair, 0)
        build(segb, dstb, relb, HCH - 1)
        addi(segb, semb)
        addw(sega, sema)
        addw(segb, semb)
        return 0
    lax.fori_loop(0, ES // HP, hist_phase, 0)
    plsc.subcore_barrier()

    WCH = WP // CHUNK

    def geti(seg, dvals, sem):
        pltpu.async_copy(den.at[seg], dvals, sem)

    def getw(seg, dvals, sem):
        pltpu.make_async_copy(den.at[seg], dvals, sem).wait()

    def w_phase(ph, _):
        base = pl.multiple_of(wid * EW + ph * WP, 8)
        pltpu.sync_copy(dst_hbm.at[pl.ds(base, WP)], dst2)
        pltpu.sync_copy(rel_hbm.at[pl.ds(base, WP)], rel2)
        pltpu.sync_copy(src_hbm.at[pl.ds(base, WP)], src2)

        def emit(dvals, i):
            for k in range(CHUNK // 16):
                ds16 = pl.ds(i * CHUNK + k * 16, 16)
                wb[ds16] = 1.0 / (dvals[pl.ds(k * 16, 16)] + EPS)
                gb[ds16] = rel2[ds16] * N + src2[ds16]

        build(sega, dst2, rel2, 0)
        geti(sega, dena, sema)

        def w_pair(k, _):
            i0 = k * 2
            build(segb, dst2, rel2, i0 + 1)
            geti(segb, denb, semb)
            getw(sega, dena, sema)
            emit(dena, i0)
            build(sega, dst2, rel2, i0 + 2)
            geti(sega, dena, sema)
            getw(segb, denb, semb)
            emit(denb, i0 + 1)
            return 0
        lax.fori_loop(0, WCH // 2, w_pair, 0)
        getw(sega, dena, sema)
        emit(dena, WCH - 1)

        pltpu.sync_copy(wb, w_hbm.at[pl.ds(base, WP)])
        pltpu.sync_copy(gb, g_hbm.at[pl.ds(base, WP)])
        return 0
    lax.fori_loop(0, EW // WP, w_phase, 0)


NCH = EW // CHUNK
NTRI = (NCH - 5) // 3


@functools.partial(
    pl.kernel,
    out_type=jax.ShapeDtypeStruct((2, NPAD, D), jnp.float32),
    mesh=_mesh,
    scratch_types=[
        pltpu.VMEM_SHARED((NPAD, D), jnp.float32),
        pltpu.VMEM((CHUNK, D), jnp.float32),
        pltpu.VMEM((CHUNK, D), jnp.float32),
        pltpu.VMEM((CHUNK, D), jnp.float32),
        pltpu.VMEM((CHUNK,), jnp.int32),
        pltpu.VMEM((CHUNK,), jnp.int32),
        pltpu.VMEM((CHUNK,), jnp.int32),
        pltpu.VMEM((CHUNK,), jnp.int32),
        pltpu.VMEM((CHUNK,), jnp.int32),
        pltpu.VMEM((CHUNK,), jnp.int32),
        pltpu.VMEM((CHUNK + 16,), jnp.float32),
        pltpu.VMEM((CHUNK + 16,), jnp.float32),
        pltpu.VMEM((CHUNK + 16,), jnp.float32),
        pltpu.SemaphoreType.DMA,
        pltpu.SemaphoreType.DMA,
        pltpu.SemaphoreType.DMA,
        pltpu.SemaphoreType.DMA,
        pltpu.SemaphoreType.DMA,
        pltpu.SemaphoreType.DMA,
        pltpu.SemaphoreType.DMA,
        pltpu.SemaphoreType.DMA,
        pltpu.SemaphoreType.DMA,
    ],
)
def _scatter(z_hbm, g_hbm, dst_hbm, w_hbm, p_hbm,
             acc, rowsa, rowsb, rowsc, ga, gb, gc, sa, sb, sc,
             wa, wb, wc, mma, mmb, mmc, mga, mgb, mgc, msa, msb, msc):
    c = lax.axis_index("c")
    s = lax.axis_index("s")
    wid = c * 16 + s
    zv = jnp.zeros((16,), jnp.float32)
    ziv = jnp.zeros((16,), jnp.int32)
    ebase = wid * EW

    A = (rowsa, ga, sa, wa, mma, mga, msa)
    B = (rowsb, gb, sb, wb, mmb, mgb, msb)
    C = (rowsc, gc, sc, wc, mmc, mgc, msc)

    def meta_issue(S, i):
        rows, g, sx, w, mm, mg, ms = S
        off = pl.multiple_of(ebase + i * CHUNK, 8)
        pltpu.async_copy(g_hbm.at[pl.ds(off, CHUNK)], g, mm)
        pltpu.async_copy(dst_hbm.at[pl.ds(off, CHUNK)], sx, mm)
        pltpu.async_copy(w_hbm.at[pl.ds(off, CHUNK)], w.at[pl.ds(0, CHUNK)], mm)

    def meta_wait(S, i):
        rows, g, sx, w, mm, mg, ms = S
        off = pl.multiple_of(ebase + i * CHUNK, 8)
        pltpu.make_async_copy(g_hbm.at[pl.ds(off, CHUNK)], g, mm).wait()
        pltpu.make_async_copy(dst_hbm.at[pl.ds(off, CHUNK)], sx, mm).wait()
        pltpu.make_async_copy(w_hbm.at[pl.ds(off, CHUNK)],
                              w.at[pl.ds(0, CHUNK)], mm).wait()

    def gather_issue(S):
        rows, g, sx, w, mm, mg, ms = S
        pltpu.async_copy(z_hbm.at[g], rows, mg)

    def gather_wait(S):
        rows, g, sx, w, mm, mg, ms = S
        pltpu.make_async_copy(z_hbm.at[g], rows, mg).wait()

    def sct_issue(S):
        rows, g, sx, w, mm, mg, ms = S
        pltpu.async_copy(rows, acc.at[sx], ms, add=True)

    def sct_wait(S):
        rows, g, sx, w, mm, mg, ms = S
        pltpu.make_async_copy(rows, acc.at[sx], ms).wait()

    def scale(S):
        rows, g, sx, w, mm, mg, ms = S

        @plsc.parallel_loop(0, CHUNK, unroll=8)
        def edge_body(e):
            wv = w[pl.ds(e, 16)]
            ws = wv[0]
            for j in range(D // 16):
                rows[e, pl.ds(j * 16, 16)] = rows[e, pl.ds(j * 16, 16)] * ws

    def zero_rows(rows):
        def zb_body(r, _):
            for j in range(D // 16):
                rows[r, pl.ds(j * 16, 16)] = zv
            return 0
        lax.fori_loop(0, CHUNK, zb_body, 0)

    zero_rows(rowsa)
    zero_rows(rowsc)
    for k in range(CHUNK // 16):
        sc[pl.ds(k * 16, 16)] = ziv
    for k in range(NPS // CHUNK):
        r0 = pl.multiple_of(s * NPS + k * CHUNK, 8)
        pltpu.sync_copy(rowsa, acc.at[pl.ds(r0, CHUNK)])
    plsc.subcore_barrier()

    sct_issue(C)
    meta_issue(A, 0)
    meta_issue(B, 1)
    meta_wait(A, 0)
    gather_issue(A)

    def step(X, Y, Z, jz, jy, jg):
        sct_wait(Z)
        meta_issue(Z, jz)
        meta_wait(Y, jy)
        gather_issue(Y)
        gather_wait(X)
        scale(X)
        sct_issue(X)

    def tri_body(t, _):
        j0 = t * 3
        step(A, B, C, j0 + 2, j0 + 1, j0)
        step(B, C, A, j0 + 3, j0 + 2, j0 + 1)
        step(C, A, B, j0 + 4, j0 + 3, j0 + 2)
        return 0
    lax.fori_loop(0, NTRI, tri_body, 0)

    j0 = NTRI * 3
    step(A, B, C, j0 + 2, j0 + 1, j0)
    step(B, C, A, j0 + 3, j0 + 2, j0 + 1)
    step(C, A, B, j0 + 4, j0 + 3, j0 + 2)
    sct_wait(C)
    meta_wait(B, j0 + 4)
    gather_issue(B)
    gather_wait(A)
    scale(A)
    sct_issue(A)
    sct_wait(A)
    gather_wait(B)
    scale(B)
    sct_issue(B)
    sct_wait(B)
    plsc.subcore_barrier()

    for k in range(NPS // CHUNK):
        r0 = pl.multiple_of(s * NPS + k * CHUNK, 8)
        pltpu.sync_copy(acc.at[pl.ds(r0, CHUNK)], p_hbm.at[c, pl.ds(r0, CHUNK)])


def _zmm_body(h_ref, w_ref, z_ref):
    z_ref[0] = jnp.dot(h_ref[...], w_ref[...],
                       preferred_element_type=jnp.float32)


_zmm = pl.pallas_call(
    _zmm_body,
    grid=(R,),
    in_specs=[
        pl.BlockSpec((N, D), lambda r: (0, 0)),
        pl.BlockSpec((D, D), lambda r: (r, 0)),
    ],
    out_specs=pl.BlockSpec((1, N, D), lambda r: (r, 0, 0)),
    out_shape=jax.ShapeDtypeStruct((R, N, D), jnp.float32),
)


def _relu_comb(p_ref, h_ref, ws_ref, b_ref, bs_ref):
    o = p_ref[0, :N, :] + p_ref[1, :N, :] + jnp.dot(
        h_ref[...], ws_ref[...], preferred_element_type=jnp.float32)
    return jnp.maximum(o + b_ref[...] + bs_ref[...], 0.0)


def _combzmm_body(p_ref, h_ref, ws_ref, b_ref, bs_ref, w2_ref,
                  h1_ref, z_ref, h1s):
    @pl.when(pl.program_id(0) == 0)
    def _():
        o = _relu_comb(p_ref, h_ref, ws_ref, b_ref, bs_ref)
        h1s[...] = o
        h1_ref[...] = o

    z_ref[0] = jnp.dot(h1s[...], w2_ref[...],
                       preferred_element_type=jnp.float32)


_combzmm = pl.pallas_call(
    _combzmm_body,
    grid=(R,),
    in_specs=[
        pl.BlockSpec((2, NPAD, D), lambda r: (0, 0, 0)),
        pl.BlockSpec((N, D), lambda r: (0, 0)),
        pl.BlockSpec((D, D), lambda r: (0, 0)),
        pl.BlockSpec((1, D), lambda r: (0, 0)),
        pl.BlockSpec((1, D), lambda r: (0, 0)),
        pl.BlockSpec((D, D), lambda r: (r, 0)),
    ],
    out_specs=(pl.BlockSpec((N, D), lambda r: (0, 0)),
               pl.BlockSpec((1, N, D), lambda r: (r, 0, 0))),
    out_shape=(jax.ShapeDtypeStruct((N, D), jnp.float32),
               jax.ShapeDtypeStruct((R, N, D), jnp.float32)),
    scratch_shapes=[pltpu.VMEM((N, D), jnp.float32)],
)


def _combine_body(p_ref, h_ref, ws_ref, b_ref, bs_ref, out_ref, gf_ref):
    o = _relu_comb(p_ref, h_ref, ws_ref, b_ref, bs_ref)
    out_ref[...] = o
    gf_ref[...] = jnp.sum(o, axis=0, keepdims=True)


_combine = pl.pallas_call(
    _combine_body,
    out_shape=(
        jax.ShapeDtypeStruct((N, D), jnp.float32),
        jax.ShapeDtypeStruct((1, D), jnp.float32),
    ),
)


def kernel(x, edge_index, edge_type, W1, b1, W1s, b1s, W2, b2, W2s, b2s):
    src = edge_index[0]
    dst = edge_index[1]
    rel = edge_type
    w1d, g1d = _prep(dst, rel, src)
    z1 = _zmm(x, W1).reshape(R * N, D)
    p1 = _scatter(z1, g1d, dst, w1d)
    h1, z2 = _combzmm(p1, x, W1s, b1.reshape(1, D), b1s.reshape(1, D), W2)
    p2 = _scatter(z2.reshape(R * N, D), g1d, dst, w1d)
    h2, gf = _combine(p2, h1, W2s, b2.reshape(1, D), b2s.reshape(1, D))
    return (gf, h2)
